# baseline (speedup 1.0000x reference)
import functools

import jax
import jax.numpy as jnp
from jax import lax
from jax.experimental import pallas as pl
from jax.experimental.pallas import tpu as pltpu
from jax.experimental.pallas import tpu_sc as plsc

_K = 128


@functools.partial(jax.jit, static_argnames=("nc", "ns"))
def _emb_gather(idx, weight, *, nc, ns):
    nw = nc * ns
    _, n_chunks, k = idx.shape
    _, d = weight.shape
    b = nw * n_chunks * k
    b_per_w = n_chunks * k

    mesh = plsc.VectorSubcoreMesh(core_axis_name="c", subcore_axis_name="s")

    @functools.partial(
        pl.kernel,
        out_type=jax.ShapeDtypeStruct((b, d), jnp.float32),
        mesh=mesh,
        scratch_types=[
            pltpu.VMEM((n_chunks, k), jnp.int32),
            pltpu.VMEM((k, d), jnp.float32),
            pltpu.SemaphoreType.DMA,
        ],
        compiler_params=pltpu.CompilerParams(use_tc_tiling_on_sc=False),
    )
    def emb_kernel(idx_hbm, table_hbm, out_hbm, idx_v, rows_v, sem):
        wid = lax.axis_index("s") * nc + lax.axis_index("c")
        base = wid * b_per_w
        pltpu.sync_copy(idx_hbm.at[wid], idx_v)

        @pl.loop(0, n_chunks)
        def _body(g):
            pltpu.async_copy(table_hbm.at[idx_v.at[g]], rows_v, sem).wait()
            pltpu.sync_copy(rows_v, out_hbm.at[pl.ds(base + g * k, k)])

    return emb_kernel(idx, weight)


def kernel(inputs, weight):
    b0, s = inputs.shape
    _, d = weight.shape
    b = b0 * s
    info = plsc.get_sparse_core_info()
    nc, ns = info.num_cores, info.num_subcores
    nw = nc * ns
    idx = inputs.reshape(nw, b // (nw * _K), _K).astype(jnp.int32)
    out = _emb_gather(idx, weight, nc=nc, ns=ns)
    return out.reshape(b0, s, d)


# speedup vs baseline: 1.1149x; 1.1149x over previous
import functools

import jax
import jax.numpy as jnp
from jax import lax
from jax.experimental import pallas as pl
from jax.experimental.pallas import tpu as pltpu
from jax.experimental.pallas import tpu_sc as plsc

_K = 128
_NBUF = 4


@functools.partial(jax.jit, static_argnames=("nc", "ns"))
def _emb_gather(idx, weight, *, nc, ns):
    nw = nc * ns
    _, n_chunks, k = idx.shape
    _, d = weight.shape
    b = nw * n_chunks * k
    b_per_w = n_chunks * k
    n_groups = n_chunks // _NBUF
    rows_per_group = _NBUF * k

    mesh = plsc.VectorSubcoreMesh(core_axis_name="c", subcore_axis_name="s")

    @functools.partial(
        pl.kernel,
        out_type=jax.ShapeDtypeStruct((b, d), jnp.float32),
        mesh=mesh,
        scratch_types=[
            pltpu.VMEM((n_chunks, k), jnp.int32),
            pltpu.VMEM((rows_per_group, d), jnp.float32),
            pltpu.VMEM((rows_per_group, d), jnp.float32),
            pltpu.SemaphoreType.DMA,
            pltpu.SemaphoreType.DMA,
        ],
        compiler_params=pltpu.CompilerParams(use_tc_tiling_on_sc=False),
    )
    def emb_kernel(idx_hbm, table_hbm, out_hbm, idx_v, rows0, rows1, sem0, sem1):
        wid = lax.axis_index("s") * nc + lax.axis_index("c")
        base = wid * b_per_w
        pltpu.sync_copy(idx_hbm.at[wid], idx_v)

        halves = ((rows0, sem0), (rows1, sem1))

        def fire(gi, h):
            rows, sem = halves[h]
            for c in range(_NBUF):
                pltpu.async_copy(
                    table_hbm.at[idx_v.at[gi * _NBUF + c]],
                    rows.at[pl.ds(c * k, k)],
                    sem,
                )

        def drain_store(gi, h):
            rows, sem = halves[h]
            for c in range(_NBUF):
                pltpu.make_async_copy(
                    table_hbm.at[idx_v.at[gi * _NBUF + c]],
                    rows.at[pl.ds(c * k, k)],
                    sem,
                ).wait()
            pltpu.sync_copy(
                rows, out_hbm.at[pl.ds(base + gi * rows_per_group, rows_per_group)]
            )

        fire(0, 0)
        fire(1, 1)

        @pl.loop(0, n_groups - 2, step=2)
        def _grp(i):
            for h in range(2):
                gi = i + h
                drain_store(gi, h)
                fire(gi + 2, h)

        for gi in (n_groups - 2, n_groups - 1):
            drain_store(gi, gi % 2)

    return emb_kernel(idx, weight)


def kernel(inputs, weight):
    b0, s = inputs.shape
    _, d = weight.shape
    b = b0 * s
    info = plsc.get_sparse_core_info()
    nc, ns = info.num_cores, info.num_subcores
    nw = nc * ns
    idx = inputs.reshape(nw, b // (nw * _K), _K).astype(jnp.int32)
    out = _emb_gather(idx, weight, nc=nc, ns=ns)
    return out.reshape(b0, s, d)


# speedup vs baseline: 1.4845x; 1.3315x over previous
import functools

import jax
import jax.numpy as jnp
from jax import lax
from jax.experimental import pallas as pl
from jax.experimental.pallas import tpu as pltpu
from jax.experimental.pallas import tpu_sc as plsc

_K = 128
_LANES = 16


@functools.partial(jax.jit, static_argnames=("nc", "ns"))
def _emb_gather(idx, weight, *, nc, ns):
    nw = nc * ns
    _, n_chunks, k = idx.shape
    _, d = weight.shape
    b = nw * n_chunks * k
    b_per_w = n_chunks * k

    mesh = plsc.VectorSubcoreMesh(core_axis_name="c", subcore_axis_name="s")

    @functools.partial(
        pl.kernel,
        out_type=jax.ShapeDtypeStruct((b, d), jnp.float32),
        mesh=mesh,
        scratch_types=[
            pltpu.VMEM((n_chunks, k), jnp.int32),
            pltpu.VMEM((k, d), jnp.float32),
            pltpu.VMEM((k, d), jnp.float32),
            pltpu.SemaphoreType.DMA,
            pltpu.SemaphoreType.DMA,
        ],
        compiler_params=pltpu.CompilerParams(use_tc_tiling_on_sc=True),
    )
    def emb_kernel(idx_hbm, table_hbm, out_hbm, idx_v, rows0, rows1, sem0, sem1):
        wid = lax.axis_index("s") * nc + lax.axis_index("c")
        base = wid * b_per_w
        pltpu.sync_copy(idx_hbm.at[wid], idx_v)

        halves = ((rows0, sem0), (rows1, sem1))

        def fire(gi, h):
            rows, sem = halves[h]
            for j16 in range(k // _LANES):
                vvec = idx_v[gi, pl.ds(j16 * _LANES, _LANES)]
                for j in range(_LANES):
                    r = j16 * _LANES + j
                    pltpu.async_copy(
                        table_hbm.at[pl.ds(vvec[j], 1)],
                        rows.at[pl.ds(r, 1)],
                        sem,
                    )

        def drain_store(gi, h):
            rows, sem = halves[h]
            for r in range(k):
                pltpu.make_async_copy(
                    table_hbm.at[pl.ds(0, 1)], rows.at[pl.ds(r, 1)], sem
                ).wait()
            pltpu.sync_copy(rows, out_hbm.at[pl.ds(base + gi * k, k)])

        fire(0, 0)
        fire(1, 1)

        @pl.loop(0, n_chunks - 2, step=2)
        def _grp(i):
            for h in range(2):
                gi = i + h
                drain_store(gi, h)
                fire(gi + 2, h)

        for gi in (n_chunks - 2, n_chunks - 1):
            drain_store(gi, gi % 2)

    return emb_kernel(idx, weight)


def kernel(inputs, weight):
    b0, s = inputs.shape
    _, d = weight.shape
    b = b0 * s
    info = plsc.get_sparse_core_info()
    nc, ns = info.num_cores, info.num_subcores
    nw = nc * ns
    idx = inputs.reshape(nw, b // (nw * _K), _K).astype(jnp.int32)
    out = _emb_gather(idx, weight, nc=nc, ns=ns)
    return out.reshape(b0, s, d)
